# baseline (speedup 1.0000x reference)
import functools

import jax
import jax.numpy as jnp
from jax import lax
from jax.experimental import pallas as pl
from jax.experimental.pallas import tpu as pltpu
from jax.experimental.pallas import tpu_sc as plsc

_VOCAB = 1000000
_EMBED = 32
_BATCH = 16384

_NC = 2
_NS = 16
_L = 16
_NW = _NC * _NS
_BPW = _BATCH // _NW

_mesh = plsc.VectorSubcoreMesh(core_axis_name="c", subcore_axis_name="s")


@functools.partial(
    pl.kernel,
    out_type=jax.ShapeDtypeStruct((_BATCH,), jnp.float32),
    mesh=_mesh,
    scratch_types=[
        pltpu.VMEM((_BPW,), jnp.int32),
        pltpu.VMEM((_BPW,), jnp.int32),
        pltpu.VMEM((_BPW, _EMBED), jnp.float32),
        pltpu.VMEM((_BPW, _EMBED), jnp.float32),
        pltpu.VMEM((_BPW,), jnp.float32),
        pltpu.SemaphoreType.DMA,
        pltpu.SemaphoreType.DMA,
    ],
    compiler_params=pltpu.CompilerParams(
        needs_layout_passes=False, use_tc_tiling_on_sc=False),
)
def _skipgram(xc_hbm, xt_hbm, ctx_hbm, tgt_hbm, out_hbm,
              xc_v, xt_v, ctx_v, tgt_v, out_v, sem_c, sem_t):
    wid = lax.axis_index("s") * _NC + lax.axis_index("c")
    base = wid * _BPW

    pltpu.sync_copy(xc_hbm.at[pl.ds(base, _BPW)], xc_v)
    pltpu.sync_copy(xt_hbm.at[pl.ds(base, _BPW)], xt_v)

    gc = pltpu.async_copy(ctx_hbm.at[xc_v], ctx_v, sem_c)
    gt = pltpu.async_copy(tgt_hbm.at[xt_v], tgt_v, sem_t)
    gc.wait()
    gt.wait()

    lane = lax.iota(jnp.int32, _L)

    def group(g, carry):
        res = jnp.zeros((_L,), jnp.float32)
        for j in range(_L):
            r = g * _L + j
            p = (ctx_v[r, pl.ds(0, _L)] * tgt_v[r, pl.ds(0, _L)]
                 + ctx_v[r, pl.ds(_L, _L)] * tgt_v[r, pl.ds(_L, _L)])
            s = lax.reduce_sum_p.bind(p, axes=(0,))
            res = jnp.where(lane == j, s, res)
        out_v[pl.ds(g * _L, _L)] = 1.0 / (1.0 + jnp.exp(-res))
        return carry

    lax.fori_loop(0, _BPW // _L, group, 0)

    pltpu.sync_copy(out_v, out_hbm.at[pl.ds(base, _BPW)])


def kernel(x, context_table, target_table):
    xc = x[:, 0].astype(jnp.int32)
    xt = x[:, 1].astype(jnp.int32)
    return _skipgram(xc, xt, context_table, target_table)
